# baseline (speedup 1.0000x reference)
import functools

import jax
import jax.numpy as jnp
import numpy as np
from jax import lax
from jax.experimental import pallas as pl
from jax.experimental.pallas import tpu as pltpu
from jax.experimental.pallas import tpu_sc as plsc

N = 10000
E = 320000
D = 128
K = 8
EPS = 1e-09

NC = 2
NS = 16
NW = NC * NS

B = 128
N_PAD = 10240
ROWS_PER_TILE = N_PAD // NS
N_CHUNKS = 80
E_TILE = N_CHUNKS * B
E_PAD = E_TILE * NW
DEG_W = 16

_MESH = plsc.VectorSubcoreMesh(core_axis_name="c", subcore_axis_name="s")


def _fill_2d(ref, rows, value):
    cols = ref.shape[1] // 16

    def body(i, _):
        r = i // cols
        cidx = i % cols
        ref[r, pl.ds(cidx * 16, 16)] = jnp.full((16,), value, jnp.float32)
        return 0

    lax.fori_loop(0, rows * cols, body, 0)


@functools.partial(
    pl.kernel,
    out_type=jax.ShapeDtypeStruct((NC, N_PAD, DEG_W), jnp.float32),
    mesh=_MESH,
    scratch_types=[
        pltpu.VMEM((B,), jnp.int32),
        pltpu.VMEM((B,), jnp.int32),
        pltpu.VMEM((B, DEG_W), jnp.float32),
        pltpu.VMEM_SHARED((N_PAD, DEG_W), jnp.float32),
        pltpu.SemaphoreType.DMA,
    ],
)
def _deg_kernel(dst_hbm, out_hbm, didx0, didx1, ones_v, acc_sh, isem):
    c = lax.axis_index("c")
    s = lax.axis_index("s")
    wid = c * NS + s
    row0 = s * ROWS_PER_TILE
    base = wid * E_TILE

    _fill_2d(ones_v, B, 0.0)
    for j in range(ROWS_PER_TILE // B):
        pltpu.sync_copy(ones_v, acc_sh.at[pl.ds(row0 + j * B, B)])
    _fill_2d(ones_v, B, 1.0)
    pltpu.sync_copy(dst_hbm.at[pl.ds(base, B)], didx0)
    plsc.subcore_barrier()

    lim = E_PAD - B

    def pair_body(g2, _):
        g = 2 * g2
        h1 = pltpu.async_copy(dst_hbm.at[pl.ds(base + (g + 1) * B, B)],
                              didx1, isem)
        pltpu.sync_copy(ones_v, acc_sh.at[didx0], add=True)
        h1.wait()
        off2 = jnp.minimum(base + (g + 2) * B, lim)
        h2 = pltpu.async_copy(dst_hbm.at[pl.ds(off2, B)], didx0, isem)
        pltpu.sync_copy(ones_v, acc_sh.at[didx1], add=True)
        h2.wait()
        return 0

    lax.fori_loop(0, N_CHUNKS // 2, pair_body, 0)
    plsc.subcore_barrier()

    pltpu.sync_copy(acc_sh.at[pl.ds(row0, ROWS_PER_TILE)],
                    out_hbm.at[c, pl.ds(row0, ROWS_PER_TILE)])


@functools.partial(
    pl.kernel,
    out_type=jax.ShapeDtypeStruct((NC, N_PAD, D), jnp.float32),
    mesh=_MESH,
    scratch_types=[
        pltpu.VMEM((B,), jnp.int32),
        pltpu.VMEM((B,), jnp.int32),
        pltpu.VMEM((B,), jnp.int32),
        pltpu.VMEM((B,), jnp.int32),
        pltpu.VMEM((B, D), jnp.float32),
        pltpu.VMEM((B, D), jnp.float32),
        pltpu.VMEM_SHARED((N_PAD, D), jnp.float32),
        pltpu.SemaphoreType.DMA,
        pltpu.SemaphoreType.DMA,
    ],
)
def _segsum_kernel(table_hbm, src_hbm, dst_hbm, out_hbm,
                   sidx0, sidx1, didx0, didx1, buf0, buf1, acc_sh,
                   gsem, isem):
    c = lax.axis_index("c")
    s = lax.axis_index("s")
    wid = c * NS + s
    row0 = s * ROWS_PER_TILE
    base = wid * E_TILE

    _fill_2d(buf0, B, 0.0)
    for j in range(ROWS_PER_TILE // B):
        pltpu.sync_copy(buf0, acc_sh.at[pl.ds(row0 + j * B, B)])
    pltpu.sync_copy(src_hbm.at[pl.ds(base, B)], sidx0)
    pltpu.sync_copy(dst_hbm.at[pl.ds(base, B)], didx0)
    plsc.subcore_barrier()

    lim = E_PAD - B

    def pair_body(g2, _):
        g = 2 * g2
        hg0 = pltpu.async_copy(table_hbm.at[sidx0], buf0, gsem)
        off1 = base + (g + 1) * B
        hs1 = pltpu.async_copy(src_hbm.at[pl.ds(off1, B)], sidx1, isem)
        hd1 = pltpu.async_copy(dst_hbm.at[pl.ds(off1, B)], didx1, isem)
        hg0.wait()
        hs1.wait()
        hd1.wait()
        hg1 = pltpu.async_copy(table_hbm.at[sidx1], buf1, gsem)
        pltpu.sync_copy(buf0, acc_sh.at[didx0], add=True)
        off2 = jnp.minimum(base + (g + 2) * B, lim)
        hs2 = pltpu.async_copy(src_hbm.at[pl.ds(off2, B)], sidx0, isem)
        hd2 = pltpu.async_copy(dst_hbm.at[pl.ds(off2, B)], didx0, isem)
        hg1.wait()
        pltpu.sync_copy(buf1, acc_sh.at[didx1], add=True)
        hs2.wait()
        hd2.wait()
        return 0

    lax.fori_loop(0, N_CHUNKS // 2, pair_body, 0)
    plsc.subcore_barrier()

    pltpu.sync_copy(acc_sh.at[pl.ds(row0, ROWS_PER_TILE)],
                    out_hbm.at[c, pl.ds(row0, ROWS_PER_TILE)])


def _norm_feat_body(d0_ref, d1_ref, feat_ref, norm_ref, f_ref):
    deg = d0_ref[...] + d1_ref[...]
    norm = lax.rsqrt(jnp.maximum(deg, 1.0))
    norm_ref[...] = norm
    f_ref[...] = feat_ref[...] * norm


def _rescale_body(p0_ref, p1_ref, norm_ref, h_ref, f2_ref):
    norm = norm_ref[...]
    h = (p0_ref[...] + p1_ref[...]) * norm
    h_ref[...] = h
    f2_ref[...] = h * norm


def _epilogue_body(q0_ref, q1_ref, norm_ref, h_ref, feat_ref,
                   wl_ref, wm_ref, wh_ref, gl_ref, gm_ref, gh_ref,
                   bias_ref, snorm_ref, out_ref):
    step = (1.0 + 2.0 * EPS) / (K - 1)
    alpha = (lax.broadcasted_iota(jnp.int32, (1, K), 1).astype(jnp.float32)
             * step - EPS)
    gl = jnp.maximum(gl_ref[...], 0.0)
    gm = jnp.maximum(gm_ref[...], 0.0)
    gh = jnp.maximum(gh_ref[...], 0.0)
    a_l = jnp.sum(alpha * gl)
    b_l = jnp.sum((1.0 - alpha) * gl)
    a_h = jnp.sum(-alpha * gh)
    b_h = jnp.sum((1.0 - alpha) * gh)
    a_m = jnp.sum(gm)
    c_m = jnp.sum(alpha * gm)

    x = feat_ref[...]
    h = h_ref[...]
    h1 = (q0_ref[...] + q1_ref[...]) * norm_ref[...]

    dn = (((1,), (1,)), ((), ()))
    o_low = lax.dot_general(a_l * h + b_l * x, wl_ref[...], dn,
                            preferred_element_type=jnp.float32)
    o_high = lax.dot_general(a_h * h + b_h * x, wh_ref[...], dn,
                             preferred_element_type=jnp.float32)
    o_mid = lax.dot_general(a_m * h1 - c_m * x, wm_ref[...], dn,
                            preferred_element_type=jnp.float32)

    def sig(v):
        return 1.0 / (1.0 + jnp.exp(-v))

    o_low = o_low * sig(o_high + o_mid)
    o_mid = o_mid * sig(o_low + o_high)
    o_high = o_high * sig(o_mid + o_low)
    out = o_low + o_mid + o_high + bias_ref[...]
    out_ref[...] = x + out * snorm_ref[...]


def kernel(feature, edge_index, snorm_n, W_low, W_mid, W_high,
           g_low, g_mid, g_high, bias):
    src = edge_index[0]
    dst = edge_index[1]
    pad = E_PAD - E
    srcp = jnp.concatenate([src, jnp.zeros((pad,), jnp.int32)])
    dstp = jnp.concatenate([dst, jnp.full((pad,), N, jnp.int32)])
    deg_parts = _deg_kernel(dstp)
    d0 = deg_parts[0, :N, 0:1]
    d1 = deg_parts[1, :N, 0:1]

    norm, f = pl.pallas_call(
        _norm_feat_body,
        out_shape=(jax.ShapeDtypeStruct((N, 1), jnp.float32),
                   jax.ShapeDtypeStruct((N, D), jnp.float32)),
    )(d0, d1, feature)

    parts1 = _segsum_kernel(f, srcp, dstp)
    h, f2 = pl.pallas_call(
        _rescale_body,
        out_shape=(jax.ShapeDtypeStruct((N, D), jnp.float32),
                   jax.ShapeDtypeStruct((N, D), jnp.float32)),
    )(parts1[0, :N], parts1[1, :N], norm)

    parts2 = _segsum_kernel(f2, srcp, dstp)

    out = pl.pallas_call(
        _epilogue_body,
        out_shape=jax.ShapeDtypeStruct((N, D), jnp.float32),
    )(parts2[0, :N], parts2[1, :N], norm, h, feature,
      W_low, W_mid, W_high,
      g_low.reshape(1, K), g_mid.reshape(1, K), g_high.reshape(1, K),
      bias.reshape(1, D), snorm_n)
    return out
